# baseline (speedup 1.0000x reference)
import jax
import jax.numpy as jnp
from jax.experimental import pallas as pl
from jax.experimental.pallas import tpu as pltpu

N = 25000
D = 128
G = 1024
E = 400000


def _head_body(accCC, accHC, accPC, accCH, accHH, accPH, gcntC, gcntH,
               WlCC, WlHC, WrC, blC, WlCH, WlHH, WrH, blH,
               Wc, bc, Wh, bh, Wf, bf, out):
    f32 = jnp.float32
    poolC = (jnp.dot(accCC[...], WlCC[...], preferred_element_type=f32)
             + jnp.dot(accHC[...], WlHC[...], preferred_element_type=f32)
             + jnp.dot(accPC[...], WrC[...], preferred_element_type=f32)
             + gcntC[...] * blC[...])
    poolH = (jnp.dot(accCH[...], WlCH[...], preferred_element_type=f32)
             + jnp.dot(accHH[...], WlHH[...], preferred_element_type=f32)
             + jnp.dot(accPH[...], WrH[...], preferred_element_type=f32)
             + gcntH[...] * blH[...])
    C_out = jnp.dot(jax.nn.relu(poolC), Wc[...], preferred_element_type=f32) + bc[...]
    H_out = jnp.dot(jax.nn.relu(poolH), Wh[...], preferred_element_type=f32) + bh[...]
    z = jax.nn.relu(jnp.concatenate([C_out, H_out], axis=1))
    out[...] = jnp.dot(z, Wf[...], preferred_element_type=f32) + bf[...]


def _head(accCC, accHC, accPC, accCH, accHH, accPH, gcntC, gcntH,
          Wl_CC, bl_CC, Wr_CC, Wl_CH, bl_CH, Wr_CH,
          Wl_HC, bl_HC, Wr_HC, Wl_HH, bl_HH, Wr_HH,
          Wc, bc, Wh, bh, Wf, bf):
    spec = pl.BlockSpec(memory_space=pltpu.ANY) if False else None
    args = (accCC, accHC, accPC, accCH, accHH, accPH,
            gcntC.reshape(G, 1), gcntH.reshape(G, 1),
            Wl_CC, Wl_HC, Wr_CC + Wr_HC, (bl_CC + bl_HC).reshape(1, D),
            Wl_CH, Wl_HH, Wr_CH + Wr_HH, (bl_CH + bl_HH).reshape(1, D),
            Wc, bc.reshape(1, 64), Wh, bh.reshape(1, 64),
            Wf, bf.reshape(1, 1))
    out = pl.pallas_call(
        _head_body,
        out_shape=jax.ShapeDtypeStruct((G, 1), jnp.float32),
    )(*args)
    return out.reshape(-1)


def _edge_acc(x_src, ei, group_dst):
    src, dst = ei[0], ei[1]
    cnt = jax.ops.segment_sum(jnp.ones((E,), jnp.float32), dst, num_segments=N)
    w = 1.0 / jnp.maximum(cnt, 1.0)
    vals = x_src[src] * w[dst][:, None]
    return jax.ops.segment_sum(vals, group_dst[dst], num_segments=G)


def kernel(x_C, x_H, ei_CC, ei_CH, ei_HC, ei_HH, C_group, H_group,
           Wl_CC, bl_CC, Wr_CC, Wl_CH, bl_CH, Wr_CH,
           Wl_HC, bl_HC, Wr_HC, Wl_HH, bl_HH, Wr_HH,
           Wc, bc, Wh, bh, Wf, bf):
    accCC = _edge_acc(x_C, ei_CC, C_group)
    accHC = _edge_acc(x_H, ei_HC, C_group)
    accCH = _edge_acc(x_C, ei_CH, H_group)
    accHH = _edge_acc(x_H, ei_HH, H_group)
    accPC = jax.ops.segment_sum(x_C, C_group, num_segments=G)
    accPH = jax.ops.segment_sum(x_H, H_group, num_segments=G)
    gcntC = jax.ops.segment_sum(jnp.ones((N,), jnp.float32), C_group, num_segments=G)
    gcntH = jax.ops.segment_sum(jnp.ones((N,), jnp.float32), H_group, num_segments=G)
    return _head(accCC, accHC, accPC, accCH, accHH, accPH, gcntC, gcntH,
                 Wl_CC, bl_CC, Wr_CC, Wl_CH, bl_CH, Wr_CH,
                 Wl_HC, bl_HC, Wr_HC, Wl_HH, bl_HH, Wr_HH,
                 Wc, bc, Wh, bh, Wf, bf)


# speedup vs baseline: 10.5650x; 10.5650x over previous
import jax
import jax.numpy as jnp
from jax import lax
from jax.experimental import pallas as pl
from jax.experimental.pallas import tpu as pltpu
from jax.experimental.pallas import tpu_sc as plsc

N = 25000
NP = 25024
D = 128
G = 1024
E = 400000

EB = 80
NB = E // 8 // EB
PB = 64
NPB = NP // PB
ACC_ROWS = 3200


def _sc_body(srcg, dstg, grp, xcat, acc_out, gcnt_out,
             grp_t, inv_t, dst_t, src_t, g_t, w_t, tmp_t, ones_t,
             pg_t, pgi_t, rows_t, cnt_sh, acc_sh, gcnt_sh):
    cid = lax.axis_index("c")
    sid = lax.axis_index("s")
    slot = sid // 8
    tis = sid % 8
    row = cid * 2 + slot
    ebase = tis * (E // 8)
    slot_base = slot * G
    ioff = slot * N

    zero16 = jnp.zeros((16,), jnp.float32)
    one16 = jnp.ones((16,), jnp.float32)

    for j in range(5):
        w_t[pl.ds(j * 16, 16)] = zero16
        ones_t[pl.ds(j * 16, 16)] = one16

    def _zrow(r, c_):
        for c in range(8):
            rows_t[r, pl.ds(c * 16, 16)] = zero16
        return c_
    lax.fori_loop(0, EB, _zrow, 0)

    for k, n in ((0, 80), (80, 80), (160, 40)):
        pltpu.sync_copy(rows_t.at[pl.ds(0, n)],
                        acc_sh.at[pl.ds(sid * (ACC_ROWS // 16) + k, n)])
    pltpu.sync_copy(w_t, gcnt_sh.at[pl.ds(sid * 80, 80)])

    nb_s = lax.select(sid < 1, 40, 39)

    def _zcnt(i, c_):
        base = (sid + i * 16) * EB
        pltpu.sync_copy(w_t, cnt_sh.at[pl.ds(base, EB)])
        return c_
    lax.fori_loop(0, nb_s, _zcnt, 0)

    plsc.subcore_barrier()

    def _count(i, c_):
        base = ebase + i * EB
        pltpu.sync_copy(dstg.at[pl.ds(row * E + base, EB)], dst_t)
        for j in range(5):
            d16 = dst_t[pl.ds(j * 16, 16)]
            g_t[pl.ds(j * 16, 16)] = d16 + ioff
        pltpu.sync_copy(ones_t, cnt_sh.at[g_t], add=True)
        return c_
    lax.fori_loop(0, NB, _count, 0)

    plsc.subcore_barrier()

    def _inv(i, c_):
        base = (sid + i * 16) * EB
        pltpu.sync_copy(cnt_sh.at[pl.ds(base, EB)], tmp_t)
        for j in range(5):
            v = tmp_t[pl.ds(j * 16, 16)]
            tmp_t[pl.ds(j * 16, 16)] = 1.0 / jnp.maximum(v, 1.0)
        pltpu.sync_copy(tmp_t, cnt_sh.at[pl.ds(base, EB)])
        return c_
    lax.fori_loop(0, nb_s, _inv, 0)

    plsc.subcore_barrier()

    pltpu.sync_copy(grp.at[pl.ds(cid * NP, NP)], grp_t)
    pltpu.sync_copy(cnt_sh.at[pl.ds(ioff, N)], inv_t)

    def _edges(i, c_):
        base = ebase + i * EB
        pltpu.sync_copy(srcg.at[pl.ds(row * E + base, EB)], src_t)
        pltpu.sync_copy(dstg.at[pl.ds(row * E + base, EB)], dst_t)
        for j in range(5):
            d16 = dst_t[pl.ds(j * 16, 16)]
            g16 = plsc.load_gather(grp_t, [d16])
            w16 = plsc.load_gather(inv_t, [d16])
            g_t[pl.ds(j * 16, 16)] = g16 + slot_base
            w_t[pl.ds(j * 16, 16)] = w16
        pltpu.sync_copy(xcat.at[src_t], rows_t)
        for j in range(5):
            w16 = w_t[pl.ds(j * 16, 16)]
            for l in range(16):
                r = j * 16 + l
                ws = w16[l]
                for c in range(8):
                    rows_t[r, pl.ds(c * 16, 16)] = rows_t[r, pl.ds(c * 16, 16)] * ws
        pltpu.sync_copy(rows_t, acc_sh.at[g_t], add=True)
        return c_
    lax.fori_loop(0, NB, _edges, 0)

    nb_p = lax.select(sid < 7, 25, 24)

    def _pool(i, c_):
        base = (sid + i * 16) * PB
        pltpu.sync_copy(grp.at[pl.ds(cid * NP + base, PB)], pg_t)
        pltpu.sync_copy(xcat.at[pl.ds(cid * NP + base, PB)],
                        rows_t.at[pl.ds(0, PB)])
        for j in range(4):
            p16 = pg_t[pl.ds(j * 16, 16)]
            pgi_t[pl.ds(j * 16, 16)] = p16 + 2 * G
        pltpu.sync_copy(rows_t.at[pl.ds(0, PB)], acc_sh.at[pgi_t], add=True)
        pltpu.sync_copy(ones_t.at[pl.ds(0, PB)], gcnt_sh.at[pg_t], add=True)
        return c_
    lax.fori_loop(0, nb_p, _pool, 0)

    plsc.subcore_barrier()

    for k, n in ((0, 96), (96, 96)):
        pltpu.sync_copy(acc_sh.at[pl.ds(sid * 192 + k, n)],
                        acc_out.at[cid, pl.ds(sid * 192 + k, n)])
    pltpu.sync_copy(gcnt_sh.at[pl.ds(sid * 64, 64)], tmp_t.at[pl.ds(0, 64)])
    pltpu.sync_copy(tmp_t.at[pl.ds(0, 64)],
                    gcnt_out.at[pl.ds(cid * G + sid * 64, 64)])


def _sc_aggregate(xcat, srcg, dstg, grp):
    mesh = plsc.VectorSubcoreMesh(core_axis_name="c", subcore_axis_name="s")
    f32, i32 = jnp.float32, jnp.int32
    return pl.kernel(
        _sc_body,
        out_type=[jax.ShapeDtypeStruct((2, 3 * G, D), f32),
                  jax.ShapeDtypeStruct((2 * G,), f32)],
        mesh=mesh,
        compiler_params=pltpu.CompilerParams(needs_layout_passes=False),
        scratch_types=[
            pltpu.VMEM((NP,), i32),
            pltpu.VMEM((N,), f32),
            pltpu.VMEM((EB,), i32),
            pltpu.VMEM((EB,), i32),
            pltpu.VMEM((EB,), i32),
            pltpu.VMEM((EB,), f32),
            pltpu.VMEM((EB,), f32),
            pltpu.VMEM((EB,), f32),
            pltpu.VMEM((PB,), i32),
            pltpu.VMEM((PB,), i32),
            pltpu.VMEM((EB, D), f32),
            pltpu.VMEM_SHARED((2 * N,), f32),
            pltpu.VMEM_SHARED((ACC_ROWS, D), f32),
            pltpu.VMEM_SHARED((1280,), f32),
        ],
    )(srcg, dstg, grp, xcat)


def _head_body(accCC, accHC, accPC, accCH, accHH, accPH, gcntC, gcntH,
               WlCC, WlHC, WrC, blC, WlCH, WlHH, WrH, blH,
               Wc, bc, Wh, bh, Wf, bf, out):
    f32 = jnp.float32
    poolC = (jnp.dot(accCC[...], WlCC[...], preferred_element_type=f32)
             + jnp.dot(accHC[...], WlHC[...], preferred_element_type=f32)
             + jnp.dot(accPC[...], WrC[...], preferred_element_type=f32)
             + gcntC[...] * blC[...])
    poolH = (jnp.dot(accCH[...], WlCH[...], preferred_element_type=f32)
             + jnp.dot(accHH[...], WlHH[...], preferred_element_type=f32)
             + jnp.dot(accPH[...], WrH[...], preferred_element_type=f32)
             + gcntH[...] * blH[...])
    C_out = jnp.dot(jax.nn.relu(poolC), Wc[...], preferred_element_type=f32) + bc[...]
    H_out = jnp.dot(jax.nn.relu(poolH), Wh[...], preferred_element_type=f32) + bh[...]
    z = jax.nn.relu(jnp.concatenate([C_out, H_out], axis=1))
    out[...] = jnp.dot(z, Wf[...], preferred_element_type=f32) + bf[...]


def _head(acc, gcnt,
          Wl_CC, bl_CC, Wr_CC, Wl_CH, bl_CH, Wr_CH,
          Wl_HC, bl_HC, Wr_HC, Wl_HH, bl_HH, Wr_HH,
          Wc, bc, Wh, bh, Wf, bf):
    a = acc.reshape(6, G, D)
    args = (a[0], a[1], a[2], a[3], a[4], a[5],
            gcnt.reshape(2, G)[0].reshape(G, 1), gcnt.reshape(2, G)[1].reshape(G, 1),
            Wl_CC, Wl_HC, Wr_CC + Wr_HC, (bl_CC + bl_HC).reshape(1, D),
            Wl_CH, Wl_HH, Wr_CH + Wr_HH, (bl_CH + bl_HH).reshape(1, D),
            Wc, bc.reshape(1, 64), Wh, bh.reshape(1, 64),
            Wf, bf.reshape(1, 1))
    out = pl.pallas_call(
        _head_body,
        out_shape=jax.ShapeDtypeStruct((G, 1), jnp.float32),
    )(*args)
    return out.reshape(-1)


def kernel(x_C, x_H, ei_CC, ei_CH, ei_HC, ei_HH, C_group, H_group,
           Wl_CC, bl_CC, Wr_CC, Wl_CH, bl_CH, Wr_CH,
           Wl_HC, bl_HC, Wr_HC, Wl_HH, bl_HH, Wr_HH,
           Wc, bc, Wh, bh, Wf, bf):
    i32 = jnp.int32
    zpad = jnp.zeros((NP - N, D), jnp.float32)
    xcat = jnp.concatenate([x_C, zpad, x_H, zpad], axis=0)
    srcg = jnp.concatenate([ei_CC[0], ei_HC[0] + NP, ei_CH[0], ei_HH[0] + NP]).astype(i32)
    dstg = jnp.concatenate([ei_CC[1], ei_HC[1], ei_CH[1], ei_HH[1]]).astype(i32)
    gpad = jnp.full((NP - N,), G, i32)
    grp = jnp.concatenate([C_group.astype(i32), gpad,
                           H_group.astype(i32), gpad])
    acc, gcnt = _sc_aggregate(xcat, srcg, dstg, grp)
    return _head(acc, gcnt,
                 Wl_CC, bl_CC, Wr_CC, Wl_CH, bl_CH, Wr_CH,
                 Wl_HC, bl_HC, Wr_HC, Wl_HH, bl_HH, Wr_HH,
                 Wc, bc, Wh, bh, Wf, bf)


# speedup vs baseline: 19.6737x; 1.8621x over previous
import jax
import jax.numpy as jnp
from jax import lax
from jax.experimental import pallas as pl
from jax.experimental.pallas import tpu as pltpu
from jax.experimental.pallas import tpu_sc as plsc

N = 25000
NP = 25024
D = 128
G = 1024
E = 400000

EB = 80
NB = 625
JB = 5
PB = 64
ACC_ROWS = 3200


def _sc_body(srcg, dstg, grp, xcat, acc_out, gcnt_out,
             grp_t, inv_t,
             dstA, srcA, gA, wA, rowsA, scA, sgA,
             dstB, srcB, gB, wB, rowsB, scB, sgB,
             tmp_t, ones_t, pg_t, pgi_t,
             cnt_sh, acc_sh, gcnt_sh,
             gsA, gsB, ssA, ssB):
    cid = lax.axis_index("c")
    sid = lax.axis_index("s")
    slot = sid // 8
    tis = sid % 8
    row = cid * 2 + slot
    ebase = row * E + tis * (E // 8)
    slot_base = slot * G
    ioff = slot * N

    zero16 = jnp.zeros((16,), jnp.float32)
    one16 = jnp.ones((16,), jnp.float32)

    for j in range(JB):
        wA[pl.ds(j * 16, 16)] = zero16
        ones_t[pl.ds(j * 16, 16)] = one16

    def _zrow(r, c_):
        for c in range(8):
            rowsA[r, pl.ds(c * 16, 16)] = zero16
        return c_
    lax.fori_loop(0, EB, _zrow, 0)

    for k, n in ((0, 80), (80, 80), (160, 40)):
        pltpu.sync_copy(rowsA.at[pl.ds(0, n)],
                        acc_sh.at[pl.ds(sid * (ACC_ROWS // 16) + k, n)])
    pltpu.sync_copy(wA, gcnt_sh.at[pl.ds(sid * 80, 80)])

    nb_s = lax.select(sid < 1, 40, 39)

    def _zcnt(i, c_):
        base = (sid + i * 16) * 80
        pltpu.sync_copy(wA, cnt_sh.at[pl.ds(base, 80)])
        return c_
    lax.fori_loop(0, nb_s, _zcnt, 0)

    plsc.subcore_barrier()

    def _cstage(i, dstb, gb, sem):
        pltpu.async_copy(dstg.at[pl.ds(ebase + i * EB, EB)], dstb, sem)

    def _cidx(dstb, gb):
        for j in range(JB):
            d16 = dstb[pl.ds(j * 16, 16)]
            gb[pl.ds(j * 16, 16)] = d16 + ioff

    def _cfire(gb, sem):
        pltpu.async_copy(ones_t, cnt_sh.at[gb], sem, add=True)

    def _cwait_stage(i, dstb, sem):
        pltpu.make_async_copy(dstg.at[pl.ds(ebase + i * EB, EB)], dstb, sem).wait()

    def _cwait_fire(gb, sem):
        pltpu.make_async_copy(ones_t, cnt_sh.at[gb], sem).wait()

    _cstage(0, dstA, gA, gsA)
    _cstage(1, dstB, gB, gsB)

    def _cpair(p, c_):
        i = p * 2
        _cwait_stage(i, dstA, gsA)

        @pl.when(p > 0)
        def _():
            _cwait_fire(gA, ssA)
        _cidx(dstA, gA)
        _cfire(gA, ssA)

        @pl.when(p < 311)
        def _():
            _cstage(i + 2, dstA, gA, gsA)
        _cwait_stage(i + 1, dstB, gsB)

        @pl.when(p > 0)
        def _():
            _cwait_fire(gB, ssB)
        _cidx(dstB, gB)
        _cfire(gB, ssB)

        @pl.when(p < 311)
        def _():
            _cstage(i + 3, dstB, gB, gsB)
        return c_
    lax.fori_loop(0, 312, _cpair, 0)
    _cwait_fire(gA, ssA)
    _cstage(624, dstA, gA, gsA)
    _cwait_stage(624, dstA, gsA)
    _cidx(dstA, gA)
    _cfire(gA, ssA)
    _cwait_fire(gB, ssB)
    _cwait_fire(gA, ssA)

    plsc.subcore_barrier()

    def _inv(i, c_):
        base = (sid + i * 16) * 80
        pltpu.sync_copy(cnt_sh.at[pl.ds(base, 80)], tmp_t)
        for j in range(5):
            v = tmp_t[pl.ds(j * 16, 16)]
            tmp_t[pl.ds(j * 16, 16)] = 1.0 / jnp.maximum(v, 1.0)
        pltpu.sync_copy(tmp_t, cnt_sh.at[pl.ds(base, 80)])
        return c_
    lax.fori_loop(0, nb_s, _inv, 0)

    plsc.subcore_barrier()

    pltpu.sync_copy(grp.at[pl.ds(cid * NP, NP)], grp_t)
    pltpu.sync_copy(cnt_sh.at[pl.ds(ioff, N)], inv_t)

    def _stage(i, dstb, srcb, gb, wb):
        pltpu.sync_copy(srcg.at[pl.ds(ebase + i * EB, EB)], srcb)
        pltpu.sync_copy(dstg.at[pl.ds(ebase + i * EB, EB)], dstb)
        for j in range(JB):
            d16 = dstb[pl.ds(j * 16, 16)]
            gb[pl.ds(j * 16, 16)] = plsc.load_gather(grp_t, [d16]) + slot_base
            wb[pl.ds(j * 16, 16)] = plsc.load_gather(inv_t, [d16])

    def _fire_g(srcb, rowsb, sem):
        pltpu.async_copy(xcat.at[srcb], rowsb, sem)

    def _wait_g(srcb, rowsb, sem):
        pltpu.make_async_copy(xcat.at[srcb], rowsb, sem).wait()

    def _scale(rowsb, scb, wb, gb, sgb):
        def _sj(j, c_):
            w16 = wb[pl.ds(j * 16, 16)]
            sgb[pl.ds(j * 16, 16)] = gb[pl.ds(j * 16, 16)]
            for l in range(16):
                r = j * 16 + l
                ws = w16[l]
                for c in range(8):
                    scb[r, pl.ds(c * 16, 16)] = rowsb[r, pl.ds(c * 16, 16)] * ws
            return c_
        lax.fori_loop(0, JB, _sj, 0)

    def _fire_s(scb, sgb, sem):
        pltpu.async_copy(scb, acc_sh.at[sgb], sem, add=True)

    def _wait_s(scb, sgb, sem):
        pltpu.make_async_copy(scb, acc_sh.at[sgb], sem).wait()

    _stage(0, dstA, srcA, gA, wA)
    _fire_g(srcA, rowsA, gsA)
    _stage(1, dstB, srcB, gB, wB)
    _fire_g(srcB, rowsB, gsB)

    def _pair(p, c_):
        i = p * 2
        _wait_g(srcA, rowsA, gsA)

        @pl.when(p > 0)
        def _():
            _wait_s(scA, sgA, ssA)
        _scale(rowsA, scA, wA, gA, sgA)
        _fire_s(scA, sgA, ssA)

        @pl.when(p < 311)
        def _():
            _stage(i + 2, dstA, srcA, gA, wA)
            _fire_g(srcA, rowsA, gsA)
        _wait_g(srcB, rowsB, gsB)

        @pl.when(p > 0)
        def _():
            _wait_s(scB, sgB, ssB)
        _scale(rowsB, scB, wB, gB, sgB)
        _fire_s(scB, sgB, ssB)

        @pl.when(p < 311)
        def _():
            _stage(i + 3, dstB, srcB, gB, wB)
            _fire_g(srcB, rowsB, gsB)
        return c_
    lax.fori_loop(0, 312, _pair, 0)
    _wait_s(scA, sgA, ssA)
    _stage(624, dstA, srcA, gA, wA)
    _fire_g(srcA, rowsA, gsA)
    _wait_g(srcA, rowsA, gsA)
    _scale(rowsA, scA, wA, gA, sgA)
    _fire_s(scA, sgA, ssA)
    _wait_s(scB, sgB, ssB)
    _wait_s(scA, sgA, ssA)

    nb_p = lax.select(sid < 7, 25, 24)

    def _pool(i, c_):
        base = (sid + i * 16) * PB
        pltpu.sync_copy(grp.at[pl.ds(cid * NP + base, PB)], pg_t)
        pltpu.sync_copy(xcat.at[pl.ds(cid * NP + base, PB)],
                        rowsA.at[pl.ds(0, PB)])
        for j in range(4):
            p16 = pg_t[pl.ds(j * 16, 16)]
            pgi_t[pl.ds(j * 16, 16)] = p16 + 2 * G
        pltpu.sync_copy(rowsA.at[pl.ds(0, PB)], acc_sh.at[pgi_t], add=True)
        pltpu.sync_copy(ones_t.at[pl.ds(0, PB)], gcnt_sh.at[pg_t], add=True)
        return c_
    lax.fori_loop(0, nb_p, _pool, 0)

    plsc.subcore_barrier()

    for k, n in ((0, 96), (96, 96)):
        pltpu.sync_copy(acc_sh.at[pl.ds(sid * 192 + k, n)],
                        acc_out.at[cid, pl.ds(sid * 192 + k, n)])
    pltpu.sync_copy(gcnt_sh.at[pl.ds(sid * 64, 64)], tmp_t.at[pl.ds(0, 64)])
    pltpu.sync_copy(tmp_t.at[pl.ds(0, 64)],
                    gcnt_out.at[pl.ds(cid * G + sid * 64, 64)])


def _sc_aggregate(srcg, dstg, grp, xcat):
    mesh = plsc.VectorSubcoreMesh(core_axis_name="c", subcore_axis_name="s")
    f32, i32 = jnp.float32, jnp.int32
    dma = pltpu.SemaphoreType.DMA
    return pl.kernel(
        _sc_body,
        out_type=[jax.ShapeDtypeStruct((2, 3 * G, D), f32),
                  jax.ShapeDtypeStruct((2 * G,), f32)],
        mesh=mesh,
        compiler_params=pltpu.CompilerParams(needs_layout_passes=False),
        scratch_types=[
            pltpu.VMEM((NP,), i32),
            pltpu.VMEM((N,), f32),
            pltpu.VMEM((EB,), i32),
            pltpu.VMEM((EB,), i32),
            pltpu.VMEM((EB,), i32),
            pltpu.VMEM((EB,), f32),
            pltpu.VMEM((EB, D), f32),
            pltpu.VMEM((EB, D), f32),
            pltpu.VMEM((EB,), i32),
            pltpu.VMEM((EB,), i32),
            pltpu.VMEM((EB,), i32),
            pltpu.VMEM((EB,), i32),
            pltpu.VMEM((EB,), f32),
            pltpu.VMEM((EB, D), f32),
            pltpu.VMEM((EB, D), f32),
            pltpu.VMEM((EB,), i32),
            pltpu.VMEM((EB,), f32),
            pltpu.VMEM((EB,), f32),
            pltpu.VMEM((PB,), i32),
            pltpu.VMEM((PB,), i32),
            pltpu.VMEM_SHARED((2 * N,), f32),
            pltpu.VMEM_SHARED((ACC_ROWS, D), f32),
            pltpu.VMEM_SHARED((1280,), f32),
            dma, dma, dma, dma,
        ],
    )(srcg, dstg, grp, xcat)


def _head_body(accCC, accHC, accPC, accCH, accHH, accPH, gcntC, gcntH,
               WlCC, WlHC, WrC, blC, WlCH, WlHH, WrH, blH,
               Wc, bc, Wh, bh, Wf, bf, out):
    f32 = jnp.float32
    poolC = (jnp.dot(accCC[...], WlCC[...], preferred_element_type=f32)
             + jnp.dot(accHC[...], WlHC[...], preferred_element_type=f32)
             + jnp.dot(accPC[...], WrC[...], preferred_element_type=f32)
             + gcntC[...] * blC[...])
    poolH = (jnp.dot(accCH[...], WlCH[...], preferred_element_type=f32)
             + jnp.dot(accHH[...], WlHH[...], preferred_element_type=f32)
             + jnp.dot(accPH[...], WrH[...], preferred_element_type=f32)
             + gcntH[...] * blH[...])
    C_out = jnp.dot(jax.nn.relu(poolC), Wc[...], preferred_element_type=f32) + bc[...]
    H_out = jnp.dot(jax.nn.relu(poolH), Wh[...], preferred_element_type=f32) + bh[...]
    z = jax.nn.relu(jnp.concatenate([C_out, H_out], axis=1))
    out[...] = jnp.dot(z, Wf[...], preferred_element_type=f32) + bf[...]


def _head(acc, gcnt,
          Wl_CC, bl_CC, Wr_CC, Wl_CH, bl_CH, Wr_CH,
          Wl_HC, bl_HC, Wr_HC, Wl_HH, bl_HH, Wr_HH,
          Wc, bc, Wh, bh, Wf, bf):
    a = acc.reshape(6, G, D)
    g2 = gcnt.reshape(2, G)
    args = (a[0], a[1], a[2], a[3], a[4], a[5],
            g2[0].reshape(G, 1), g2[1].reshape(G, 1),
            Wl_CC, Wl_HC, Wr_CC + Wr_HC, (bl_CC + bl_HC).reshape(1, D),
            Wl_CH, Wl_HH, Wr_CH + Wr_HH, (bl_CH + bl_HH).reshape(1, D),
            Wc, bc.reshape(1, 64), Wh, bh.reshape(1, 64),
            Wf, bf.reshape(1, 1))
    out = pl.pallas_call(
        _head_body,
        out_shape=jax.ShapeDtypeStruct((G, 1), jnp.float32),
    )(*args)
    return out.reshape(-1)


def kernel(x_C, x_H, ei_CC, ei_CH, ei_HC, ei_HH, C_group, H_group,
           Wl_CC, bl_CC, Wr_CC, Wl_CH, bl_CH, Wr_CH,
           Wl_HC, bl_HC, Wr_HC, Wl_HH, bl_HH, Wr_HH,
           Wc, bc, Wh, bh, Wf, bf):
    i32 = jnp.int32
    zpad = jnp.zeros((NP - N, D), jnp.float32)
    xcat = jnp.concatenate([x_C, zpad, x_H, zpad], axis=0)
    srcg = jnp.concatenate([ei_CC[0], ei_HC[0] + NP, ei_CH[0], ei_HH[0] + NP]).astype(i32)
    dstg = jnp.concatenate([ei_CC[1], ei_HC[1], ei_CH[1], ei_HH[1]]).astype(i32)
    gpad = jnp.full((NP - N,), G, i32)
    grp = jnp.concatenate([C_group.astype(i32), gpad,
                           H_group.astype(i32), gpad])
    acc, gcnt = _sc_aggregate(srcg, dstg, grp, xcat)
    return _head(acc, gcnt,
                 Wl_CC, bl_CC, Wr_CC, Wl_CH, bl_CH, Wr_CH,
                 Wl_HC, bl_HC, Wr_HC, Wl_HH, bl_HH, Wr_HH,
                 Wc, bc, Wh, bh, Wf, bf)


# speedup vs baseline: 25.7190x; 1.3073x over previous
import jax
import jax.numpy as jnp
from jax import lax
from jax.experimental import pallas as pl
from jax.experimental.pallas import tpu as pltpu
from jax.experimental.pallas import tpu_sc as plsc

N = 25000
NP = 25024
D = 128
G = 1024
E = 400000

EB = 80
NB = 625
JB = 5
PB = 64
ACC_ROWS = 3200


def _sc_body(srcg, dstg, grp, xcat, acc_out, gcnt_out,
             grp_t, inv_t,
             dstA, srcA, gA, wA, rowsA, scA, sgA,
             dstB, srcB, gB, wB, rowsB, scB, sgB,
             tmp_t, ones_t, pg_t, pgi_t,
             cnt_sh, acc_sh, gcnt_sh,
             gsA, gsB, ssA, ssB, stA, stB):
    cid = lax.axis_index("c")
    sid = lax.axis_index("s")
    slot = sid // 8
    tis = sid % 8
    row = cid * 2 + slot
    ebase = row * E + tis * (E // 8)
    slot_base = slot * G
    ioff = slot * N

    zero16 = jnp.zeros((16,), jnp.float32)
    one16 = jnp.ones((16,), jnp.float32)

    for j in range(JB):
        wA[pl.ds(j * 16, 16)] = zero16
        ones_t[pl.ds(j * 16, 16)] = one16

    def _zrow(r, c_):
        for c in range(8):
            rowsA[r, pl.ds(c * 16, 16)] = zero16
        return c_
    lax.fori_loop(0, EB, _zrow, 0)

    for k, n in ((0, 80), (80, 80), (160, 40)):
        pltpu.sync_copy(rowsA.at[pl.ds(0, n)],
                        acc_sh.at[pl.ds(sid * (ACC_ROWS // 16) + k, n)])
    pltpu.sync_copy(wA, gcnt_sh.at[pl.ds(sid * 80, 80)])

    nb_s = lax.select(sid < 1, 40, 39)

    def _zcnt(i, c_):
        base = (sid + i * 16) * 80
        pltpu.sync_copy(wA, cnt_sh.at[pl.ds(base, 80)])
        return c_
    lax.fori_loop(0, nb_s, _zcnt, 0)

    plsc.subcore_barrier()

    def _cstage(i, dstb, gb, sem):
        pltpu.async_copy(dstg.at[pl.ds(ebase + i * EB, EB)], dstb, sem)

    def _cidx(dstb, gb):
        for j in range(JB):
            d16 = dstb[pl.ds(j * 16, 16)]
            gb[pl.ds(j * 16, 16)] = d16 + ioff

    def _cfire(gb, sem):
        pltpu.async_copy(ones_t, cnt_sh.at[gb], sem, add=True)

    def _cwait_stage(i, dstb, sem):
        pltpu.make_async_copy(dstg.at[pl.ds(ebase + i * EB, EB)], dstb, sem).wait()

    def _cwait_fire(gb, sem):
        pltpu.make_async_copy(ones_t, cnt_sh.at[gb], sem).wait()

    _cstage(0, dstA, gA, gsA)
    _cstage(1, dstB, gB, gsB)

    def _cpair(p, c_):
        i = p * 2
        _cwait_stage(i, dstA, gsA)

        @pl.when(p > 0)
        def _():
            _cwait_fire(gA, ssA)
        _cidx(dstA, gA)
        _cfire(gA, ssA)

        @pl.when(p < 311)
        def _():
            _cstage(i + 2, dstA, gA, gsA)
        _cwait_stage(i + 1, dstB, gsB)

        @pl.when(p > 0)
        def _():
            _cwait_fire(gB, ssB)
        _cidx(dstB, gB)
        _cfire(gB, ssB)

        @pl.when(p < 311)
        def _():
            _cstage(i + 3, dstB, gB, gsB)
        return c_
    lax.fori_loop(0, 312, _cpair, 0)
    _cwait_fire(gA, ssA)
    _cstage(624, dstA, gA, gsA)
    _cwait_stage(624, dstA, gsA)
    _cidx(dstA, gA)
    _cfire(gA, ssA)
    _cwait_fire(gB, ssB)
    _cwait_fire(gA, ssA)

    plsc.subcore_barrier()

    def _inv(i, c_):
        base = (sid + i * 16) * 80
        pltpu.sync_copy(cnt_sh.at[pl.ds(base, 80)], tmp_t)
        for j in range(5):
            v = tmp_t[pl.ds(j * 16, 16)]
            tmp_t[pl.ds(j * 16, 16)] = 1.0 / jnp.maximum(v, 1.0)
        pltpu.sync_copy(tmp_t, cnt_sh.at[pl.ds(base, 80)])
        return c_
    lax.fori_loop(0, nb_s, _inv, 0)

    plsc.subcore_barrier()

    pltpu.sync_copy(grp.at[pl.ds(cid * NP, NP)], grp_t)
    pltpu.sync_copy(cnt_sh.at[pl.ds(ioff, N)], inv_t)

    def _fire_stage(i, dstb, srcb, sem):
        pltpu.async_copy(srcg.at[pl.ds(ebase + i * EB, EB)], srcb, sem)
        pltpu.async_copy(dstg.at[pl.ds(ebase + i * EB, EB)], dstb, sem)

    def _wait_stage(i, dstb, srcb, sem):
        pltpu.make_async_copy(srcg.at[pl.ds(ebase + i * EB, EB)], srcb, sem).wait()
        pltpu.make_async_copy(dstg.at[pl.ds(ebase + i * EB, EB)], dstb, sem).wait()

    def _lookup(dstb, gb, wb):
        for j in range(JB):
            d16 = dstb[pl.ds(j * 16, 16)]
            gb[pl.ds(j * 16, 16)] = plsc.load_gather(grp_t, [d16]) + slot_base
            wb[pl.ds(j * 16, 16)] = plsc.load_gather(inv_t, [d16])

    def _fire_g(srcb, rowsb, sem):
        pltpu.async_copy(xcat.at[srcb], rowsb, sem)

    def _wait_g(srcb, rowsb, sem):
        pltpu.make_async_copy(xcat.at[srcb], rowsb, sem).wait()

    def _scale(rowsb, scb, wb, gb, sgb):
        def _sj(j, c_):
            w16 = wb[pl.ds(j * 16, 16)]
            sgb[pl.ds(j * 16, 16)] = gb[pl.ds(j * 16, 16)]
            for l in range(16):
                r = j * 16 + l
                ws = w16[l]
                for c in range(8):
                    scb[r, pl.ds(c * 16, 16)] = rowsb[r, pl.ds(c * 16, 16)] * ws
            return c_
        lax.fori_loop(0, JB, _sj, 0)

    def _fire_s(scb, sgb, sem):
        pltpu.async_copy(scb, acc_sh.at[sgb], sem, add=True)

    def _wait_s(scb, sgb, sem):
        pltpu.make_async_copy(scb, acc_sh.at[sgb], sem).wait()

    _fire_stage(0, dstA, srcA, gsA)
    _fire_stage(1, dstB, srcB, gsB)
    _wait_stage(0, dstA, srcA, gsA)
    _lookup(dstA, gA, wA)
    _fire_g(srcA, rowsA, gsA)
    _wait_stage(1, dstB, srcB, gsB)
    _lookup(dstB, gB, wB)
    _fire_g(srcB, rowsB, gsB)

    def _pair(p, c_):
        i = p * 2
        _wait_g(srcA, rowsA, gsA)
        _fire_stage(i + 2, dstA, srcA, stA)

        @pl.when(p > 0)
        def _():
            _wait_s(scA, sgA, ssA)
        _scale(rowsA, scA, wA, gA, sgA)
        _fire_s(scA, sgA, ssA)
        _wait_stage(i + 2, dstA, srcA, stA)
        _lookup(dstA, gA, wA)
        _fire_g(srcA, rowsA, gsA)
        _wait_g(srcB, rowsB, gsB)

        @pl.when(p < 311)
        def _():
            _fire_stage(i + 3, dstB, srcB, stB)

        @pl.when(p > 0)
        def _():
            _wait_s(scB, sgB, ssB)
        _scale(rowsB, scB, wB, gB, sgB)
        _fire_s(scB, sgB, ssB)

        @pl.when(p < 311)
        def _():
            _wait_stage(i + 3, dstB, srcB, stB)
            _lookup(dstB, gB, wB)
            _fire_g(srcB, rowsB, gsB)
        return c_
    lax.fori_loop(0, 312, _pair, 0)
    _wait_g(srcA, rowsA, gsA)
    _wait_s(scA, sgA, ssA)
    _scale(rowsA, scA, wA, gA, sgA)
    _fire_s(scA, sgA, ssA)
    _wait_s(scB, sgB, ssB)
    _wait_s(scA, sgA, ssA)

    nb_p = lax.select(sid < 7, 25, 24)

    def _pool(i, c_):
        base = (sid + i * 16) * PB
        pltpu.sync_copy(grp.at[pl.ds(cid * NP + base, PB)], pg_t)
        pltpu.sync_copy(xcat.at[pl.ds(cid * NP + base, PB)],
                        rowsA.at[pl.ds(0, PB)])
        for j in range(4):
            p16 = pg_t[pl.ds(j * 16, 16)]
            pgi_t[pl.ds(j * 16, 16)] = p16 + 2 * G
        pltpu.sync_copy(rowsA.at[pl.ds(0, PB)], acc_sh.at[pgi_t], add=True)
        pltpu.sync_copy(ones_t.at[pl.ds(0, PB)], gcnt_sh.at[pg_t], add=True)
        return c_
    lax.fori_loop(0, nb_p, _pool, 0)

    plsc.subcore_barrier()

    for k, n in ((0, 96), (96, 96)):
        pltpu.sync_copy(acc_sh.at[pl.ds(sid * 192 + k, n)],
                        acc_out.at[cid, pl.ds(sid * 192 + k, n)])
    pltpu.sync_copy(gcnt_sh.at[pl.ds(sid * 64, 64)], tmp_t.at[pl.ds(0, 64)])
    pltpu.sync_copy(tmp_t.at[pl.ds(0, 64)],
                    gcnt_out.at[pl.ds(cid * G + sid * 64, 64)])


def _sc_aggregate(srcg, dstg, grp, xcat):
    mesh = plsc.VectorSubcoreMesh(core_axis_name="c", subcore_axis_name="s")
    f32, i32 = jnp.float32, jnp.int32
    dma = pltpu.SemaphoreType.DMA
    return pl.kernel(
        _sc_body,
        out_type=[jax.ShapeDtypeStruct((2, 3 * G, D), f32),
                  jax.ShapeDtypeStruct((2 * G,), f32)],
        mesh=mesh,
        compiler_params=pltpu.CompilerParams(needs_layout_passes=False),
        scratch_types=[
            pltpu.VMEM((NP,), i32),
            pltpu.VMEM((N,), f32),
            pltpu.VMEM((EB,), i32),
            pltpu.VMEM((EB,), i32),
            pltpu.VMEM((EB,), i32),
            pltpu.VMEM((EB,), f32),
            pltpu.VMEM((EB, D), f32),
            pltpu.VMEM((EB, D), f32),
            pltpu.VMEM((EB,), i32),
            pltpu.VMEM((EB,), i32),
            pltpu.VMEM((EB,), i32),
            pltpu.VMEM((EB,), i32),
            pltpu.VMEM((EB,), f32),
            pltpu.VMEM((EB, D), f32),
            pltpu.VMEM((EB, D), f32),
            pltpu.VMEM((EB,), i32),
            pltpu.VMEM((EB,), f32),
            pltpu.VMEM((EB,), f32),
            pltpu.VMEM((PB,), i32),
            pltpu.VMEM((PB,), i32),
            pltpu.VMEM_SHARED((2 * N,), f32),
            pltpu.VMEM_SHARED((ACC_ROWS, D), f32),
            pltpu.VMEM_SHARED((1280,), f32),
            dma, dma, dma, dma, dma, dma,
        ],
    )(srcg, dstg, grp, xcat)


def _head_body(accCC, accHC, accPC, accCH, accHH, accPH, gcntC, gcntH,
               WlCC, WlHC, WrC, blC, WlCH, WlHH, WrH, blH,
               Wc, bc, Wh, bh, Wf, bf, out):
    f32 = jnp.float32
    poolC = (jnp.dot(accCC[...], WlCC[...], preferred_element_type=f32)
             + jnp.dot(accHC[...], WlHC[...], preferred_element_type=f32)
             + jnp.dot(accPC[...], WrC[...], preferred_element_type=f32)
             + gcntC[...] * blC[...])
    poolH = (jnp.dot(accCH[...], WlCH[...], preferred_element_type=f32)
             + jnp.dot(accHH[...], WlHH[...], preferred_element_type=f32)
             + jnp.dot(accPH[...], WrH[...], preferred_element_type=f32)
             + gcntH[...] * blH[...])
    C_out = jnp.dot(jax.nn.relu(poolC), Wc[...], preferred_element_type=f32) + bc[...]
    H_out = jnp.dot(jax.nn.relu(poolH), Wh[...], preferred_element_type=f32) + bh[...]
    z = jax.nn.relu(jnp.concatenate([C_out, H_out], axis=1))
    out[...] = jnp.dot(z, Wf[...], preferred_element_type=f32) + bf[...]


def _head(acc, gcnt,
          Wl_CC, bl_CC, Wr_CC, Wl_CH, bl_CH, Wr_CH,
          Wl_HC, bl_HC, Wr_HC, Wl_HH, bl_HH, Wr_HH,
          Wc, bc, Wh, bh, Wf, bf):
    a = acc.reshape(6, G, D)
    g2 = gcnt.reshape(2, G)
    args = (a[0], a[1], a[2], a[3], a[4], a[5],
            g2[0].reshape(G, 1), g2[1].reshape(G, 1),
            Wl_CC, Wl_HC, Wr_CC + Wr_HC, (bl_CC + bl_HC).reshape(1, D),
            Wl_CH, Wl_HH, Wr_CH + Wr_HH, (bl_CH + bl_HH).reshape(1, D),
            Wc, bc.reshape(1, 64), Wh, bh.reshape(1, 64),
            Wf, bf.reshape(1, 1))
    out = pl.pallas_call(
        _head_body,
        out_shape=jax.ShapeDtypeStruct((G, 1), jnp.float32),
    )(*args)
    return out.reshape(-1)


def kernel(x_C, x_H, ei_CC, ei_CH, ei_HC, ei_HH, C_group, H_group,
           Wl_CC, bl_CC, Wr_CC, Wl_CH, bl_CH, Wr_CH,
           Wl_HC, bl_HC, Wr_HC, Wl_HH, bl_HH, Wr_HH,
           Wc, bc, Wh, bh, Wf, bf):
    i32 = jnp.int32
    zpad = jnp.zeros((NP - N, D), jnp.float32)
    xcat = jnp.concatenate([x_C, zpad, x_H, zpad], axis=0)
    srcg = jnp.concatenate([ei_CC[0], ei_HC[0] + NP, ei_CH[0], ei_HH[0] + NP]).astype(i32)
    dstg = jnp.concatenate([ei_CC[1], ei_HC[1], ei_CH[1], ei_HH[1]]).astype(i32)
    gpad = jnp.full((NP - N,), G, i32)
    grp = jnp.concatenate([C_group.astype(i32), gpad,
                           H_group.astype(i32), gpad])
    acc, gcnt = _sc_aggregate(srcg, dstg, grp, xcat)
    return _head(acc, gcnt,
                 Wl_CC, bl_CC, Wr_CC, Wl_CH, bl_CH, Wr_CH,
                 Wl_HC, bl_HC, Wr_HC, Wl_HH, bl_HH, Wr_HH,
                 Wc, bc, Wh, bh, Wf, bf)
